# Initial kernel scaffold; baseline (speedup 1.0000x reference)
import jax
import jax.numpy as jnp
from jax.experimental import pallas as pl


def kernel(x, batch, W1, b1, prelu_a, W2, b2):
    raise NotImplementedError("write your pallas kernel here")


# speedup vs baseline: 3.5961x; 3.5961x over previous
import functools

import jax
import jax.numpy as jnp
from jax import lax
from jax.experimental import pallas as pl
from jax.experimental.pallas import tpu as pltpu
from jax.experimental.pallas import tpu_sc as plsc

NC = 2
NS = 16
L = 16
NW = NC * NS

N = 100000
D = 128
DC = D // L
G = 1024
GT = 1152
GROWS = GT // NS

RW = 3136
BLK = 224
NBLK = RW // BLK
TAIL_FULL = (N - 31 * RW) // BLK
TAIL_REM = (N - 31 * RW) % BLK

_mesh = plsc.VectorSubcoreMesh(core_axis_name="c", subcore_axis_name="s")


@functools.partial(
    pl.kernel,
    out_type=(
        jax.ShapeDtypeStruct((NC, GT * D), jnp.float32),
        jax.ShapeDtypeStruct((NC, GT * L), jnp.float32),
    ),
    mesh=_mesh,
    scratch_types=[
        pltpu.VMEM((BLK * D,), jnp.float32),
        pltpu.VMEM((BLK,), jnp.int32),
        pltpu.VMEM((D,), jnp.float32),
        pltpu.VMEM((L,), jnp.float32),
        pltpu.VMEM((D,), jnp.int32),
        pltpu.VMEM((L,), jnp.int32),
        pltpu.VMEM((GROWS * D,), jnp.float32),
        pltpu.VMEM((GROWS * L,), jnp.float32),
        pltpu.VMEM_SHARED((GT * D,), jnp.float32),
        pltpu.VMEM_SHARED((GT * L,), jnp.float32),
    ],
)
def _segment_sums(x_hbm, b_hbm, sums_hbm, cnts_hbm,
                  xb, ib, stage_row, stage_cnt, stage_ri, stage_ci,
                  obuf, cbuf, sums_sh, cnts_sh):
    sid = lax.axis_index("s")
    cid = lax.axis_index("c")
    wid = cid * NS + sid
    lane = lax.iota(jnp.int32, L)
    zv = jnp.zeros((L,), jnp.float32)

    for k in range(DC):
        stage_row[pl.ds(k * L, L)] = zv

    def _zchunk(i, _):
        base = sid * (GROWS * D) + i * D
        for k in range(DC):
            stage_ri[pl.ds(k * L, L)] = base + k * L + lane
        pltpu.sync_copy(stage_row, sums_sh.at[stage_ri])
        return 0
    lax.fori_loop(0, GROWS, _zchunk, 0)

    def _zcchunk(i, _):
        base = sid * (GROWS * L) + i * D
        for k in range(DC):
            stage_ri[pl.ds(k * L, L)] = base + k * L + lane
        pltpu.sync_copy(stage_row, cnts_sh.at[stage_ri])
        return 0
    lax.fori_loop(0, GROWS * L // D, _zcchunk, 0)
    plsc.subcore_barrier()

    def _flush(accs, cntv, cur):
        base_s = cur * D
        for k in range(DC):
            stage_row[pl.ds(k * L, L)] = accs[k]
            stage_ri[pl.ds(k * L, L)] = base_s + (k * L) + lane
        stage_cnt[pl.ds(0, L)] = cntv
        stage_ci[pl.ds(0, L)] = cur * L + lane
        pltpu.sync_copy(stage_row, sums_sh.at[stage_ri], add=True)
        pltpu.sync_copy(stage_cnt, cnts_sh.at[stage_ci], add=True)

    def _group_body(g, carry):
        ids16 = ib[pl.ds(g * L, L)]
        for j in range(L):
            accs, cntv, cur = carry
            rid = ids16[j]
            change = rid != cur

            @pl.when(change)
            def _(accs=accs, cntv=cntv, cur=cur):
                _flush(accs, cntv, cur)

            r = g * L + j
            row = [xb[pl.ds(r * D + k * L, L)] for k in range(DC)]
            naccs = tuple(
                jnp.where(change, row[k], accs[k] + row[k])
                for k in range(DC)
            )
            ncnt = jnp.where(change, jnp.float32(1.0),
                             cntv + jnp.float32(1.0))
            carry = (naccs, ncnt, rid)
        return carry

    def _run_span(base0, nblk, tail_rows):
        carry0 = (
            tuple(jnp.zeros((L,), jnp.float32) for _ in range(DC)),
            jnp.zeros((L,), jnp.float32),
            jnp.int32(0),
        )

        def _blk_body(t, carry):
            base = base0 + t * BLK
            pltpu.sync_copy(x_hbm.at[pl.ds(base * D, BLK * D)], xb)
            pltpu.sync_copy(b_hbm.at[pl.ds(base, BLK)], ib)
            return lax.fori_loop(0, BLK // L, _group_body, carry)

        carry = lax.fori_loop(0, nblk, _blk_body, carry0)
        if tail_rows:
            base = base0 + nblk * BLK
            pltpu.sync_copy(x_hbm.at[pl.ds(base * D, tail_rows * D)],
                            xb.at[pl.ds(0, tail_rows * D)])
            pltpu.sync_copy(b_hbm.at[pl.ds(base, tail_rows)],
                            ib.at[pl.ds(0, tail_rows)])
            carry = lax.fori_loop(0, tail_rows // L, _group_body, carry)
        accs, cntv, cur = carry
        _flush(accs, cntv, cur)

    @pl.when(wid < NW - 1)
    def _():
        _run_span(wid * RW, NBLK, 0)

    @pl.when(wid == NW - 1)
    def _():
        _run_span((NW - 1) * RW, TAIL_FULL, TAIL_REM)

    plsc.subcore_barrier()

    for j in range(NS):
        @pl.when(sid == j)
        def _(j=j):
            pltpu.sync_copy(sums_sh.at[pl.ds(j * GROWS * D, GROWS * D)], obuf)
            pltpu.sync_copy(obuf,
                            sums_hbm.at[cid, pl.ds(j * GROWS * D, GROWS * D)])
            pltpu.sync_copy(cnts_sh.at[pl.ds(j * GROWS * L, GROWS * L)], cbuf)
            pltpu.sync_copy(cbuf,
                            cnts_hbm.at[cid, pl.ds(j * GROWS * L, GROWS * L)])


def _mlp_body(sums_ref, cnts_ref, w1_ref, b1_ref, a_ref, w2_ref, b2_ref,
              out_ref):
    total = sums_ref[0] + sums_ref[1]
    cnt = cnts_ref[0] + cnts_ref[1]
    cnt1 = jnp.maximum(cnt[:, 0:1], 1.0)
    mean = total / cnt1
    h = lax.dot_general(mean, w1_ref[...], (((1,), (1,)), ((), ())),
                        preferred_element_type=jnp.float32)
    h = h + b1_ref[...]
    h = jnp.where(h >= 0, h, a_ref[...] * h)
    s = lax.dot_general(h, w2_ref[...], (((1,), (1,)), ((), ())),
                        preferred_element_type=jnp.float32)
    out_ref[...] = jax.nn.sigmoid(s + b2_ref[...])


def _attention_scores(sums, cnts, W1, b1, prelu_a, W2, b2):
    return pl.pallas_call(
        _mlp_body,
        out_shape=jax.ShapeDtypeStruct((GT, L), jnp.float32),
    )(sums, cnts, W1, b1, prelu_a, W2, b2)


@functools.partial(
    pl.kernel,
    out_type=jax.ShapeDtypeStruct((N * D,), jnp.float32),
    mesh=_mesh,
    scratch_types=[
        pltpu.VMEM((BLK * D,), jnp.float32),
        pltpu.VMEM((BLK,), jnp.int32),
        pltpu.VMEM((GT * L,), jnp.float32),
    ],
)
def _scale_nodes(x_hbm, b_hbm, s_hbm, out_hbm, xb, ib, sv):
    sid = lax.axis_index("s")
    cid = lax.axis_index("c")
    wid = cid * NS + sid
    pltpu.sync_copy(s_hbm, sv)

    def _group_body(g, _):
        ids16 = ib[pl.ds(g * L, L)]
        for j in range(L):
            rid = ids16[j]
            srow = sv[pl.ds(rid * L, L)]
            r = g * L + j
            for k in range(DC):
                o = r * D + k * L
                xb[pl.ds(o, L)] = xb[pl.ds(o, L)] * srow
        return 0

    def _do_block(base, nrows):
        pltpu.sync_copy(x_hbm.at[pl.ds(base * D, nrows * D)],
                        xb.at[pl.ds(0, nrows * D)])
        pltpu.sync_copy(b_hbm.at[pl.ds(base, nrows)],
                        ib.at[pl.ds(0, nrows)])
        lax.fori_loop(0, nrows // L, _group_body, 0)
        pltpu.sync_copy(xb.at[pl.ds(0, nrows * D)],
                        out_hbm.at[pl.ds(base * D, nrows * D)])

    @pl.when(wid < NW - 1)
    def _():
        def _blk_body(t, _):
            _do_block(wid * RW + t * BLK, BLK)
            return 0
        lax.fori_loop(0, NBLK, _blk_body, 0)

    @pl.when(wid == NW - 1)
    def _():
        base0 = (NW - 1) * RW

        def _blk_body(t, _):
            _do_block(base0 + t * BLK, BLK)
            return 0
        lax.fori_loop(0, TAIL_FULL, _blk_body, 0)
        _do_block(base0 + TAIL_FULL * BLK, TAIL_REM)


def kernel(x, batch, W1, b1, prelu_a, W2, b2):
    bi = batch.astype(jnp.int32)
    xf = jnp.reshape(x, (-1,))
    sums, cnts = _segment_sums(xf, bi)
    scores = _attention_scores(
        jnp.reshape(sums, (NC, GT, D)),
        jnp.reshape(cnts, (NC, GT, L)),
        W1,
        jnp.reshape(b1, (1, -1)),
        jnp.reshape(jnp.asarray(prelu_a, jnp.float32), (1, 1)),
        jnp.tile(W2, (L, 1)),
        jnp.reshape(b2, (1, 1)),
    )
    return jnp.reshape(_scale_nodes(xf, bi, jnp.reshape(scores, (-1,))),
                       (N, D))


# speedup vs baseline: 5.2101x; 1.4488x over previous
import functools

import jax
import jax.numpy as jnp
from jax import lax
from jax.experimental import pallas as pl
from jax.experimental.pallas import tpu as pltpu
from jax.experimental.pallas import tpu_sc as plsc

NC = 2
NS = 16
L = 16
NW = NC * NS

N = 100000
D = 128
DC = D // L
G = 1024
GT = 1152
GROWS = GT // NS

RW = 3136
BLK = 224
NBLK = RW // BLK
TAIL_FULL = (N - 31 * RW) // BLK
TAIL_REM = (N - 31 * RW) % BLK

_mesh = plsc.VectorSubcoreMesh(core_axis_name="c", subcore_axis_name="s")


@functools.partial(
    pl.kernel,
    out_type=(
        jax.ShapeDtypeStruct((NC, GT * D), jnp.float32),
        jax.ShapeDtypeStruct((NC, GT * L), jnp.float32),
    ),
    mesh=_mesh,
    scratch_types=[
        pltpu.VMEM((BLK * D,), jnp.float32),
        pltpu.VMEM((BLK * D,), jnp.float32),
        pltpu.VMEM((RW,), jnp.int32),
        pltpu.VMEM((D,), jnp.float32),
        pltpu.VMEM((L,), jnp.float32),
        pltpu.VMEM((D,), jnp.int32),
        pltpu.VMEM((L,), jnp.int32),
        pltpu.VMEM((GROWS * D,), jnp.float32),
        pltpu.VMEM((GROWS * L,), jnp.float32),
        pltpu.VMEM_SHARED((GT * D,), jnp.float32),
        pltpu.VMEM_SHARED((GT * L,), jnp.float32),
        pltpu.SemaphoreType.DMA,
        pltpu.SemaphoreType.DMA,
    ],
)
def _segment_sums(x_hbm, b_hbm, sums_hbm, cnts_hbm,
                  xb0, xb1, ib, stage_row, stage_cnt, stage_ri, stage_ci,
                  obuf, cbuf, sums_sh, cnts_sh, sem0, sem1):
    sid = lax.axis_index("s")
    cid = lax.axis_index("c")
    wid = cid * NS + sid
    lane = lax.iota(jnp.int32, L)
    zv = jnp.zeros((L,), jnp.float32)

    for k in range(DC):
        stage_row[pl.ds(k * L, L)] = zv

    def _zchunk(i, _):
        base = sid * (GROWS * D) + i * D
        for k in range(DC):
            stage_ri[pl.ds(k * L, L)] = base + k * L + lane
        pltpu.sync_copy(stage_row, sums_sh.at[stage_ri])
        return 0
    lax.fori_loop(0, GROWS, _zchunk, 0)

    def _zcchunk(i, _):
        base = sid * (GROWS * L) + i * D
        for k in range(DC):
            stage_ri[pl.ds(k * L, L)] = base + k * L + lane
        pltpu.sync_copy(stage_row, cnts_sh.at[stage_ri])
        return 0
    lax.fori_loop(0, GROWS * L // D, _zcchunk, 0)
    plsc.subcore_barrier()

    def _flush(cnt, cur):
        base_s = cur * D
        for k in range(DC):
            stage_ri[pl.ds(k * L, L)] = base_s + (k * L) + lane
        stage_cnt[pl.ds(0, L)] = jnp.full((L,), cnt, jnp.float32)
        stage_ci[pl.ds(0, L)] = cur * L + lane
        pltpu.sync_copy(stage_row, sums_sh.at[stage_ri], add=True)
        pltpu.sync_copy(stage_cnt, cnts_sh.at[stage_ci], add=True)

    def _make_group_body(buf, t):
        def _group_body(g, carry):
            ids16 = ib[pl.ds(t * BLK + g * L, L)]
            uniform = ids16[L - 1] == carry[1]

            def _fast(carry):
                cnt, cur = carry
                for k in range(DC):
                    v = [buf[pl.ds((g * L + j) * D + k * L, L)]
                         for j in range(L)]
                    while len(v) > 1:
                        v = [v[i] + v[i + 1] for i in range(0, len(v), 2)]
                    o = k * L
                    stage_row[pl.ds(o, L)] = stage_row[pl.ds(o, L)] + v[0]
                return (cnt + jnp.float32(L), cur)

            def _slow(carry):
                cnt, cur = carry
                for j in range(L):
                    rid = ids16[j]
                    change = rid != cur

                    @pl.when(change)
                    def _(cnt=cnt, cur=cur):
                        _flush(cnt, cur)

                    r = g * L + j
                    for k in range(DC):
                        o = k * L
                        row = buf[pl.ds(r * D + o, L)]
                        stage_row[pl.ds(o, L)] = jnp.where(
                            change, row, stage_row[pl.ds(o, L)] + row)
                    cnt = jnp.where(change, jnp.float32(1.0),
                                    cnt + jnp.float32(1.0))
                    cur = rid
                return (cnt, cur)

            return lax.cond(uniform, _fast, _slow, carry)
        return _group_body

    def _run_span(base0, nblk, tail_rows):
        nids = nblk * BLK + tail_rows
        pltpu.sync_copy(b_hbm.at[pl.ds(base0, nids)], ib.at[pl.ds(0, nids)])

        def _x_slice(t):
            return x_hbm.at[pl.ds((base0 + t * BLK) * D, BLK * D)]

        pltpu.async_copy(_x_slice(0), xb0, sem0)
        carry0 = (jnp.float32(0.0), jnp.int32(0))

        def _pair_body(p, carry):
            t0 = 2 * p
            pltpu.async_copy(_x_slice(t0 + 1), xb1, sem1)
            pltpu.make_async_copy(_x_slice(t0), xb0, sem0).wait()
            carry = lax.fori_loop(0, BLK // L,
                                  _make_group_body(xb0, t0), carry)

            @pl.when(p < nblk // 2 - 1)
            def _():
                pltpu.async_copy(_x_slice(t0 + 2), xb0, sem0)

            pltpu.make_async_copy(_x_slice(t0 + 1), xb1, sem1).wait()
            carry = lax.fori_loop(0, BLK // L,
                                  _make_group_body(xb1, t0 + 1), carry)
            return carry

        carry = lax.fori_loop(0, nblk // 2, _pair_body, carry0)
        if tail_rows:
            base = base0 + nblk * BLK
            pltpu.sync_copy(x_hbm.at[pl.ds(base * D, tail_rows * D)],
                            xb0.at[pl.ds(0, tail_rows * D)])
            carry = lax.fori_loop(0, tail_rows // L,
                                  _make_group_body(xb0, nblk), carry)
        cnt, cur = carry
        _flush(cnt, cur)

    @pl.when(wid < NW - 1)
    def _():
        _run_span(wid * RW, NBLK, 0)

    @pl.when(wid == NW - 1)
    def _():
        _run_span((NW - 1) * RW, TAIL_FULL, TAIL_REM)

    plsc.subcore_barrier()

    for j in range(NS):
        @pl.when(sid == j)
        def _(j=j):
            pltpu.sync_copy(sums_sh.at[pl.ds(j * GROWS * D, GROWS * D)], obuf)
            pltpu.sync_copy(obuf,
                            sums_hbm.at[cid, pl.ds(j * GROWS * D, GROWS * D)])
            pltpu.sync_copy(cnts_sh.at[pl.ds(j * GROWS * L, GROWS * L)], cbuf)
            pltpu.sync_copy(cbuf,
                            cnts_hbm.at[cid, pl.ds(j * GROWS * L, GROWS * L)])


def _mlp_body(sums_ref, cnts_ref, w1_ref, b1_ref, a_ref, w2_ref, b2_ref,
              out_ref):
    total = sums_ref[0] + sums_ref[1]
    cnt = cnts_ref[0] + cnts_ref[1]
    cnt1 = jnp.maximum(cnt[:, 0:1], 1.0)
    mean = total / cnt1
    h = lax.dot_general(mean, w1_ref[...], (((1,), (1,)), ((), ())),
                        preferred_element_type=jnp.float32)
    h = h + b1_ref[...]
    h = jnp.where(h >= 0, h, a_ref[...] * h)
    s = lax.dot_general(h, w2_ref[...], (((1,), (1,)), ((), ())),
                        preferred_element_type=jnp.float32)
    out_ref[...] = jax.nn.sigmoid(s + b2_ref[...])


def _attention_scores(sums, cnts, W1, b1, prelu_a, W2, b2):
    return pl.pallas_call(
        _mlp_body,
        out_shape=jax.ShapeDtypeStruct((GT, L), jnp.float32),
    )(sums, cnts, W1, b1, prelu_a, W2, b2)


BLKC = 112
NBC = RW // BLKC

@functools.partial(
    pl.kernel,
    out_type=jax.ShapeDtypeStruct((N * D,), jnp.float32),
    mesh=_mesh,
    scratch_types=[
        pltpu.VMEM((BLKC * D,), jnp.float32),
        pltpu.VMEM((BLKC * D,), jnp.float32),
        pltpu.VMEM((BLKC * D,), jnp.float32),
        pltpu.VMEM((BLKC * D,), jnp.float32),
        pltpu.VMEM((BLKC,), jnp.int32),
        pltpu.VMEM((BLKC,), jnp.int32),
        pltpu.VMEM((G * L,), jnp.float32),
        pltpu.SemaphoreType.DMA,
        pltpu.SemaphoreType.DMA,
        pltpu.SemaphoreType.DMA,
        pltpu.SemaphoreType.DMA,
    ],
)
def _scale_nodes(x_hbm, b_hbm, s_hbm, out_hbm,
                 in0, in1, out0, out1, ids0, ids1, sv,
                 semL0, semL1, semS0, semS1):
    sid = lax.axis_index("s")
    cid = lax.axis_index("c")
    wid = cid * NS + sid
    pltpu.sync_copy(s_hbm.at[pl.ds(0, G * L)], sv)
    base0 = wid * RW

    def _base(t):
        return jnp.minimum(base0 + t * BLKC, N - BLKC)

    def _start_load(t, inb, idb, sem):
        b = _base(t)
        pltpu.async_copy(x_hbm.at[pl.ds(b * D, BLKC * D)], inb, sem)
        pltpu.async_copy(b_hbm.at[pl.ds(b, BLKC)], idb, sem)

    def _wait_load(t, inb, idb, sem):
        b = _base(t)
        pltpu.make_async_copy(x_hbm.at[pl.ds(b * D, BLKC * D)], inb,
                              sem).wait()
        pltpu.make_async_copy(b_hbm.at[pl.ds(b, BLKC)], idb, sem).wait()

    def _start_store(t, outb, sem):
        b = _base(t)
        pltpu.async_copy(outb, out_hbm.at[pl.ds(b * D, BLKC * D)], sem)

    def _wait_store(t, outb, sem):
        b = _base(t)
        pltpu.make_async_copy(outb, out_hbm.at[pl.ds(b * D, BLKC * D)],
                              sem).wait()

    def _compute(inb, idb, outb):
        def _group_body(g, _):
            ids16 = idb[pl.ds(g * L, L)]
            for j in range(L):
                rid = ids16[j]
                srow = sv[pl.ds(rid * L, L)]
                o = (g * L + j) * D
                for k in range(DC):
                    outb[pl.ds(o + k * L, L)] = (
                        inb[pl.ds(o + k * L, L)] * srow)
            return 0
        lax.fori_loop(0, BLKC // L, _group_body, 0)

    _start_load(0, in0, ids0, semL0)

    def _pair_body(p, _):
        t0 = 2 * p
        _start_load(t0 + 1, in1, ids1, semL1)
        _wait_load(t0, in0, ids0, semL0)

        @pl.when(p >= 1)
        def _():
            _wait_store(t0 - 2, out0, semS0)

        _compute(in0, ids0, out0)
        _start_store(t0, out0, semS0)

        @pl.when(p < NBC // 2 - 1)
        def _():
            _start_load(t0 + 2, in0, ids0, semL0)

        _wait_load(t0 + 1, in1, ids1, semL1)

        @pl.when(p >= 1)
        def _():
            _wait_store(t0 - 1, out1, semS1)

        _compute(in1, ids1, out1)
        _start_store(t0 + 1, out1, semS1)
        return 0

    lax.fori_loop(0, NBC // 2, _pair_body, 0)
    _wait_store(NBC - 2, out0, semS0)
    _wait_store(NBC - 1, out1, semS1)


def kernel(x, batch, W1, b1, prelu_a, W2, b2):
    bi = batch.astype(jnp.int32)
    xf = jnp.reshape(x, (-1,))
    sums, cnts = _segment_sums(xf, bi)
    scores = _attention_scores(
        jnp.reshape(sums, (NC, GT, D)),
        jnp.reshape(cnts, (NC, GT, L)),
        W1,
        jnp.reshape(b1, (1, -1)),
        jnp.reshape(jnp.asarray(prelu_a, jnp.float32), (1, 1)),
        jnp.tile(W2, (L, 1)),
        jnp.reshape(b2, (1, 1)),
    )
    return jnp.reshape(_scale_nodes(xf, bi, jnp.reshape(scores, (-1,))),
                       (N, D))


# speedup vs baseline: 5.3758x; 1.0318x over previous
import functools

import jax
import jax.numpy as jnp
from jax import lax
from jax.experimental import pallas as pl
from jax.experimental.pallas import tpu as pltpu
from jax.experimental.pallas import tpu_sc as plsc

NC = 2
NS = 16
L = 16
NW = NC * NS

N = 100000
D = 128
DC = D // L
G = 1024
GT = 1152
GROWS = GT // NS

RW = 3136
BLK = 224
NBLK = RW // BLK
TAIL_FULL = (N - 31 * RW) // BLK
TAIL_REM = (N - 31 * RW) % BLK

_mesh = plsc.VectorSubcoreMesh(core_axis_name="c", subcore_axis_name="s")


@functools.partial(
    pl.kernel,
    out_type=(
        jax.ShapeDtypeStruct((NC, GT * D), jnp.float32),
        jax.ShapeDtypeStruct((NC, GT * L), jnp.float32),
    ),
    mesh=_mesh,
    scratch_types=[
        pltpu.VMEM((BLK * D,), jnp.float32),
        pltpu.VMEM((BLK * D,), jnp.float32),
        pltpu.VMEM((RW,), jnp.int32),
        pltpu.VMEM((D,), jnp.float32),
        pltpu.VMEM((L,), jnp.float32),
        pltpu.VMEM((D,), jnp.int32),
        pltpu.VMEM((L,), jnp.int32),
        pltpu.VMEM((GROWS * D,), jnp.float32),
        pltpu.VMEM((GROWS * L,), jnp.float32),
        pltpu.VMEM_SHARED((GT * D,), jnp.float32),
        pltpu.VMEM_SHARED((GT * L,), jnp.float32),
        pltpu.SemaphoreType.DMA,
        pltpu.SemaphoreType.DMA,
        [pltpu.VMEM((D,), jnp.int32)] * 4,
        [pltpu.SemaphoreType.DMA] * 4,
    ],
)
def _segment_sums(x_hbm, b_hbm, sums_hbm, cnts_hbm,
                  xb0, xb1, ib, stage_row, stage_cnt, stage_ri, stage_ci,
                  obuf, cbuf, sums_sh, cnts_sh, sem0, sem1, zri, zsem):
    sid = lax.axis_index("s")
    cid = lax.axis_index("c")
    wid = cid * NS + sid
    lane = lax.iota(jnp.int32, L)
    zv = jnp.zeros((L,), jnp.float32)

    for k in range(DC):
        stage_row[pl.ds(k * L, L)] = zv

    NZS = G * D // NS // D
    NZC = G * L // NS // D
    for i in range(NZS + NZC):
        slot = i % 4
        if i < NZS:
            dst, base = sums_sh, sid * (G * D // NS) + i * D
        else:
            dst, base = cnts_sh, sid * (G * L // NS) + (i - NZS) * D
        if i >= 4:
            pltpu.make_async_copy(stage_row, dst.at[zri[slot]],
                                  zsem[slot]).wait()
        for k in range(DC):
            zri[slot][pl.ds(k * L, L)] = base + k * L + lane
        pltpu.async_copy(stage_row, dst.at[zri[slot]], zsem[slot])
    for slot in range(4):
        pltpu.make_async_copy(stage_row, sums_sh.at[zri[slot]],
                              zsem[slot]).wait()
    plsc.subcore_barrier()

    def _flush(cnt, cur):
        base_s = cur * D
        for k in range(DC):
            stage_ri[pl.ds(k * L, L)] = base_s + (k * L) + lane
        stage_cnt[pl.ds(0, L)] = jnp.full((L,), cnt, jnp.float32)
        stage_ci[pl.ds(0, L)] = cur * L + lane
        pltpu.sync_copy(stage_row, sums_sh.at[stage_ri], add=True)
        pltpu.sync_copy(stage_cnt, cnts_sh.at[stage_ci], add=True)

    def _make_group_body(buf, t):
        def _group_body(g, carry):
            ids16 = ib[pl.ds(t * BLK + g * L, L)]
            uniform = ids16[L - 1] == carry[1]

            def _fast(carry):
                cnt, cur = carry
                for k in range(DC):
                    v = [buf[pl.ds((g * L + j) * D + k * L, L)]
                         for j in range(L)]
                    while len(v) > 1:
                        v = [v[i] + v[i + 1] for i in range(0, len(v), 2)]
                    o = k * L
                    stage_row[pl.ds(o, L)] = stage_row[pl.ds(o, L)] + v[0]
                return (cnt + jnp.float32(L), cur)

            def _slow(carry):
                cnt, cur = carry
                for j in range(L):
                    rid = ids16[j]
                    change = rid != cur

                    @pl.when(change)
                    def _(cnt=cnt, cur=cur):
                        _flush(cnt, cur)

                    r = g * L + j
                    for k in range(DC):
                        o = k * L
                        row = buf[pl.ds(r * D + o, L)]
                        stage_row[pl.ds(o, L)] = jnp.where(
                            change, row, stage_row[pl.ds(o, L)] + row)
                    cnt = jnp.where(change, jnp.float32(1.0),
                                    cnt + jnp.float32(1.0))
                    cur = rid
                return (cnt, cur)

            return lax.cond(uniform, _fast, _slow, carry)
        return _group_body

    def _run_span(base0, nblk, tail_rows):
        nids = nblk * BLK + tail_rows
        pltpu.sync_copy(b_hbm.at[pl.ds(base0, nids)], ib.at[pl.ds(0, nids)])

        def _x_slice(t):
            return x_hbm.at[pl.ds((base0 + t * BLK) * D, BLK * D)]

        pltpu.async_copy(_x_slice(0), xb0, sem0)
        carry0 = (jnp.float32(0.0), jnp.int32(0))

        def _pair_body(p, carry):
            t0 = 2 * p
            pltpu.async_copy(_x_slice(t0 + 1), xb1, sem1)
            pltpu.make_async_copy(_x_slice(t0), xb0, sem0).wait()
            carry = lax.fori_loop(0, BLK // L,
                                  _make_group_body(xb0, t0), carry)

            @pl.when(p < nblk // 2 - 1)
            def _():
                pltpu.async_copy(_x_slice(t0 + 2), xb0, sem0)

            pltpu.make_async_copy(_x_slice(t0 + 1), xb1, sem1).wait()
            carry = lax.fori_loop(0, BLK // L,
                                  _make_group_body(xb1, t0 + 1), carry)
            return carry

        carry = lax.fori_loop(0, nblk // 2, _pair_body, carry0)
        if tail_rows:
            base = base0 + nblk * BLK
            pltpu.sync_copy(x_hbm.at[pl.ds(base * D, tail_rows * D)],
                            xb0.at[pl.ds(0, tail_rows * D)])
            carry = lax.fori_loop(0, tail_rows // L,
                                  _make_group_body(xb0, nblk), carry)
        cnt, cur = carry
        _flush(cnt, cur)

    @pl.when(wid < NW - 1)
    def _():
        _run_span(wid * RW, NBLK, 0)

    @pl.when(wid == NW - 1)
    def _():
        _run_span((NW - 1) * RW, TAIL_FULL, TAIL_REM)

    plsc.subcore_barrier()

    for j in range(NS):
        @pl.when(sid == j)
        def _(j=j):
            pltpu.sync_copy(sums_sh.at[pl.ds(j * GROWS * D, GROWS * D)], obuf)
            pltpu.sync_copy(obuf,
                            sums_hbm.at[cid, pl.ds(j * GROWS * D, GROWS * D)])
            pltpu.sync_copy(cnts_sh.at[pl.ds(j * GROWS * L, GROWS * L)], cbuf)
            pltpu.sync_copy(cbuf,
                            cnts_hbm.at[cid, pl.ds(j * GROWS * L, GROWS * L)])


def _mlp_body(sums_ref, cnts_ref, w1_ref, b1_ref, a_ref, w2_ref, b2_ref,
              out_ref):
    total = sums_ref[0] + sums_ref[1]
    cnt = cnts_ref[0] + cnts_ref[1]
    cnt1 = jnp.maximum(cnt[:, 0:1], 1.0)
    mean = total / cnt1
    h = lax.dot_general(mean, w1_ref[...], (((1,), (1,)), ((), ())),
                        preferred_element_type=jnp.float32)
    h = h + b1_ref[...]
    h = jnp.where(h >= 0, h, a_ref[...] * h)
    s = lax.dot_general(h, w2_ref[...], (((1,), (1,)), ((), ())),
                        preferred_element_type=jnp.float32)
    out_ref[...] = jax.nn.sigmoid(s + b2_ref[...])


def _attention_scores(sums, cnts, W1, b1, prelu_a, W2, b2):
    return pl.pallas_call(
        _mlp_body,
        out_shape=jax.ShapeDtypeStruct((GT, L), jnp.float32),
    )(sums, cnts, W1, b1, prelu_a, W2, b2)


BLKC = 112
NBC = RW // BLKC

@functools.partial(
    pl.kernel,
    out_type=jax.ShapeDtypeStruct((N * D,), jnp.float32),
    mesh=_mesh,
    scratch_types=[
        pltpu.VMEM((BLKC * D,), jnp.float32),
        pltpu.VMEM((BLKC * D,), jnp.float32),
        pltpu.VMEM((BLKC * D,), jnp.float32),
        pltpu.VMEM((BLKC * D,), jnp.float32),
        pltpu.VMEM((BLKC,), jnp.int32),
        pltpu.VMEM((BLKC,), jnp.int32),
        pltpu.VMEM((G * L,), jnp.float32),
        pltpu.SemaphoreType.DMA,
        pltpu.SemaphoreType.DMA,
        pltpu.SemaphoreType.DMA,
        pltpu.SemaphoreType.DMA,
    ],
)
def _scale_nodes(x_hbm, b_hbm, s_hbm, out_hbm,
                 in0, in1, out0, out1, ids0, ids1, sv,
                 semL0, semL1, semS0, semS1):
    sid = lax.axis_index("s")
    cid = lax.axis_index("c")
    wid = cid * NS + sid
    pltpu.sync_copy(s_hbm.at[pl.ds(0, G * L)], sv)
    base0 = wid * RW

    def _base(t):
        return jnp.minimum(base0 + t * BLKC, N - BLKC)

    def _start_load(t, inb, idb, sem):
        b = _base(t)
        pltpu.async_copy(x_hbm.at[pl.ds(b * D, BLKC * D)], inb, sem)
        pltpu.async_copy(b_hbm.at[pl.ds(b, BLKC)], idb, sem)

    def _wait_load(t, inb, idb, sem):
        b = _base(t)
        pltpu.make_async_copy(x_hbm.at[pl.ds(b * D, BLKC * D)], inb,
                              sem).wait()
        pltpu.make_async_copy(b_hbm.at[pl.ds(b, BLKC)], idb, sem).wait()

    def _start_store(t, outb, sem):
        b = _base(t)
        pltpu.async_copy(outb, out_hbm.at[pl.ds(b * D, BLKC * D)], sem)

    def _wait_store(t, outb, sem):
        b = _base(t)
        pltpu.make_async_copy(outb, out_hbm.at[pl.ds(b * D, BLKC * D)],
                              sem).wait()

    def _compute(inb, idb, outb):
        def _group_body(g, _):
            ids16 = idb[pl.ds(g * L, L)]
            for j in range(L):
                rid = ids16[j]
                srow = sv[pl.ds(rid * L, L)]
                o = (g * L + j) * D
                for k in range(DC):
                    outb[pl.ds(o + k * L, L)] = (
                        inb[pl.ds(o + k * L, L)] * srow)
            return 0
        lax.fori_loop(0, BLKC // L, _group_body, 0)

    _start_load(0, in0, ids0, semL0)

    def _pair_body(p, _):
        t0 = 2 * p
        _start_load(t0 + 1, in1, ids1, semL1)
        _wait_load(t0, in0, ids0, semL0)

        @pl.when(p >= 1)
        def _():
            _wait_store(t0 - 2, out0, semS0)

        _compute(in0, ids0, out0)
        _start_store(t0, out0, semS0)

        @pl.when(p < NBC // 2 - 1)
        def _():
            _start_load(t0 + 2, in0, ids0, semL0)

        _wait_load(t0 + 1, in1, ids1, semL1)

        @pl.when(p >= 1)
        def _():
            _wait_store(t0 - 1, out1, semS1)

        _compute(in1, ids1, out1)
        _start_store(t0 + 1, out1, semS1)
        return 0

    lax.fori_loop(0, NBC // 2, _pair_body, 0)
    _wait_store(NBC - 2, out0, semS0)
    _wait_store(NBC - 1, out1, semS1)


def kernel(x, batch, W1, b1, prelu_a, W2, b2):
    bi = batch.astype(jnp.int32)
    xf = jnp.reshape(x, (-1,))
    sums, cnts = _segment_sums(xf, bi)
    scores = _attention_scores(
        jnp.reshape(sums, (NC, GT, D)),
        jnp.reshape(cnts, (NC, GT, L)),
        W1,
        jnp.reshape(b1, (1, -1)),
        jnp.reshape(jnp.asarray(prelu_a, jnp.float32), (1, 1)),
        jnp.tile(W2, (L, 1)),
        jnp.reshape(b2, (1, 1)),
    )
    return jnp.reshape(_scale_nodes(xf, bi, jnp.reshape(scores, (-1,))),
                       (N, D))


# speedup vs baseline: 5.6963x; 1.0596x over previous
import functools

import jax
import jax.numpy as jnp
from jax import lax
from jax.experimental import pallas as pl
from jax.experimental.pallas import tpu as pltpu
from jax.experimental.pallas import tpu_sc as plsc

NC = 2
NS = 16
L = 16
NW = NC * NS

N = 100000
D = 128
DC = D // L
G = 1024
GT = 1152
GROWS = GT // NS

RW = 3136
BLK = 224
NBLK = RW // BLK
TAIL_FULL = (N - 31 * RW) // BLK
TAIL_REM = (N - 31 * RW) % BLK

_mesh = plsc.VectorSubcoreMesh(core_axis_name="c", subcore_axis_name="s")


@functools.partial(
    pl.kernel,
    out_type=(
        jax.ShapeDtypeStruct((NC, GT * D), jnp.float32),
        jax.ShapeDtypeStruct((NC, GT * L), jnp.float32),
    ),
    mesh=_mesh,
    scratch_types=[
        pltpu.VMEM((BLK * D,), jnp.float32),
        pltpu.VMEM((BLK * D,), jnp.float32),
        pltpu.VMEM((RW + L,), jnp.int32),
        pltpu.VMEM((D,), jnp.float32),
        pltpu.VMEM((L,), jnp.float32),
        pltpu.VMEM((D,), jnp.int32),
        pltpu.VMEM((L,), jnp.int32),
        pltpu.VMEM((GROWS * D,), jnp.float32),
        pltpu.VMEM((GROWS * L,), jnp.float32),
        pltpu.VMEM_SHARED((GT * D,), jnp.float32),
        pltpu.VMEM_SHARED((GT * L,), jnp.float32),
        pltpu.SemaphoreType.DMA,
        pltpu.SemaphoreType.DMA,
        [pltpu.VMEM((D,), jnp.int32)] * 4,
        [pltpu.SemaphoreType.DMA] * 4,
        pltpu.VMEM((D,), jnp.float32),
        pltpu.VMEM((L,), jnp.float32),
        pltpu.VMEM((D,), jnp.int32),
        pltpu.VMEM((L,), jnp.int32),
        pltpu.SemaphoreType.DMA,
    ],
)
def _segment_sums(x_hbm, b_hbm, sums_hbm, cnts_hbm,
                  xb0, xb1, ib, stage_row, stage_cnt, stage_ri, stage_ci,
                  obuf, cbuf, sums_sh, cnts_sh, sem0, sem1, zri, zsem,
                  fbuf, fcnt, fri, fci, fsem):
    sid = lax.axis_index("s")
    cid = lax.axis_index("c")
    wid = cid * NS + sid
    lane = lax.iota(jnp.int32, L)
    zv = jnp.zeros((L,), jnp.float32)

    for k in range(DC):
        stage_row[pl.ds(k * L, L)] = zv

    NZS = G * D // NS // D
    NZC = G * L // NS // D

    def _zloop(it, _):
        for slot in range(4):
            i = it * 4 + slot

            @pl.when(it >= 1)
            def _():
                pltpu.make_async_copy(stage_row, sums_sh.at[zri[slot]],
                                      zsem[slot]).wait()

            is_sum = i < NZS
            base = jnp.where(is_sum, sid * (G * D // NS) + i * D,
                             sid * (G * L // NS) + (i - NZS) * D)
            for k in range(DC):
                zri[slot][pl.ds(k * L, L)] = base + k * L + lane

            @pl.when(is_sum)
            def _():
                pltpu.async_copy(stage_row, sums_sh.at[zri[slot]],
                                 zsem[slot])

            @pl.when(jnp.logical_not(is_sum))
            def _():
                pltpu.async_copy(stage_row, cnts_sh.at[zri[slot]],
                                 zsem[slot])
        return 0
    lax.fori_loop(0, (NZS + NZC) // 4, _zloop, 0)
    for slot in range(4):
        pltpu.make_async_copy(stage_row, sums_sh.at[zri[slot]],
                              zsem[slot]).wait()

    for k in range(DC):
        fri[pl.ds(k * L, L)] = k * L + lane
        fbuf[pl.ds(k * L, L)] = zv
    fci[pl.ds(0, L)] = lane
    fcnt[pl.ds(0, L)] = zv
    pltpu.async_copy(fbuf, sums_sh.at[fri], fsem, add=True)
    pltpu.async_copy(fcnt, cnts_sh.at[fci], fsem, add=True)
    plsc.subcore_barrier()

    def _drain_flush():
        pltpu.make_async_copy(fbuf, sums_sh.at[fri], fsem).wait()
        pltpu.make_async_copy(fcnt, cnts_sh.at[fci], fsem).wait()

    def _flush(cnt, cur):
        _drain_flush()
        base_s = cur * D
        for k in range(DC):
            fri[pl.ds(k * L, L)] = base_s + (k * L) + lane
            fbuf[pl.ds(k * L, L)] = stage_row[pl.ds(k * L, L)]
        fcnt[pl.ds(0, L)] = jnp.full((L,), cnt, jnp.float32)
        fci[pl.ds(0, L)] = cur * L + lane
        pltpu.async_copy(fbuf, sums_sh.at[fri], fsem, add=True)
        pltpu.async_copy(fcnt, cnts_sh.at[fci], fsem, add=True)

    def _make_group_body(buf, t):
        def _group_body(g, carry):
            ids16 = ib[pl.ds(t * BLK + g * L, L)]
            uniform = ids16[L - 1] == carry[1]

            def _fast(carry):
                cnt, cur = carry
                for k in range(DC):
                    v = [buf[pl.ds((g * L + j) * D + k * L, L)]
                         for j in range(L)]
                    while len(v) > 1:
                        v = [v[i] + v[i + 1] for i in range(0, len(v), 2)]
                    o = k * L
                    stage_row[pl.ds(o, L)] = stage_row[pl.ds(o, L)] + v[0]
                return (cnt + jnp.float32(L), cur)

            def _slow(carry):
                def _row_body(j, c):
                    cnt, cur = c
                    rid = ib[pl.ds(t * BLK + g * L + j, L)][0]
                    change = rid != cur

                    @pl.when(change)
                    def _(cnt=cnt, cur=cur):
                        _flush(cnt, cur)

                    r = g * L + j
                    for k in range(DC):
                        o = k * L
                        row = buf[pl.ds(r * D + o, L)]
                        stage_row[pl.ds(o, L)] = jnp.where(
                            change, row, stage_row[pl.ds(o, L)] + row)
                    cnt = jnp.where(change, jnp.float32(1.0),
                                    cnt + jnp.float32(1.0))
                    return (cnt, rid)
                return lax.fori_loop(0, L, _row_body, carry)

            return lax.cond(uniform, _fast, _slow, carry)
        return _group_body

    def _run_span(base0, nblk, tail_rows):
        nids = nblk * BLK + tail_rows
        pltpu.sync_copy(b_hbm.at[pl.ds(base0, nids)], ib.at[pl.ds(0, nids)])

        def _x_slice(t):
            return x_hbm.at[pl.ds((base0 + t * BLK) * D, BLK * D)]

        pltpu.async_copy(_x_slice(0), xb0, sem0)
        carry0 = (jnp.float32(0.0), jnp.int32(0))

        def _pair_body(p, carry):
            t0 = 2 * p
            pltpu.async_copy(_x_slice(t0 + 1), xb1, sem1)
            pltpu.make_async_copy(_x_slice(t0), xb0, sem0).wait()
            carry = lax.fori_loop(0, BLK // L,
                                  _make_group_body(xb0, t0), carry)

            @pl.when(p < nblk // 2 - 1)
            def _():
                pltpu.async_copy(_x_slice(t0 + 2), xb0, sem0)

            pltpu.make_async_copy(_x_slice(t0 + 1), xb1, sem1).wait()
            carry = lax.fori_loop(0, BLK // L,
                                  _make_group_body(xb1, t0 + 1), carry)
            return carry

        carry = lax.fori_loop(0, nblk // 2, _pair_body, carry0)
        if tail_rows:
            base = base0 + nblk * BLK
            pltpu.sync_copy(x_hbm.at[pl.ds(base * D, tail_rows * D)],
                            xb0.at[pl.ds(0, tail_rows * D)])
            carry = lax.fori_loop(0, tail_rows // L,
                                  _make_group_body(xb0, nblk), carry)
        cnt, cur = carry
        _flush(cnt, cur)
        _drain_flush()

    @pl.when(wid < NW - 1)
    def _():
        _run_span(wid * RW, NBLK, 0)

    @pl.when(wid == NW - 1)
    def _():
        _run_span((NW - 1) * RW, TAIL_FULL, TAIL_REM)

    plsc.subcore_barrier()

    for j in range(NS):
        @pl.when(sid == j)
        def _(j=j):
            pltpu.sync_copy(sums_sh.at[pl.ds(j * GROWS * D, GROWS * D)], obuf)
            pltpu.sync_copy(obuf,
                            sums_hbm.at[cid, pl.ds(j * GROWS * D, GROWS * D)])
            pltpu.sync_copy(cnts_sh.at[pl.ds(j * GROWS * L, GROWS * L)], cbuf)
            pltpu.sync_copy(cbuf,
                            cnts_hbm.at[cid, pl.ds(j * GROWS * L, GROWS * L)])


def _mlp_body(sums_ref, cnts_ref, w1_ref, b1_ref, a_ref, w2_ref, b2_ref,
              out_ref):
    total = sums_ref[0] + sums_ref[1]
    cnt = cnts_ref[0] + cnts_ref[1]
    cnt1 = jnp.maximum(cnt[:, 0:1], 1.0)
    mean = total / cnt1
    h = lax.dot_general(mean, w1_ref[...], (((1,), (1,)), ((), ())),
                        preferred_element_type=jnp.float32)
    h = h + b1_ref[...]
    h = jnp.where(h >= 0, h, a_ref[...] * h)
    s = lax.dot_general(h, w2_ref[...], (((1,), (1,)), ((), ())),
                        preferred_element_type=jnp.float32)
    out_ref[...] = jax.nn.sigmoid(s + b2_ref[...])


def _attention_scores(sums, cnts, W1, b1, prelu_a, W2, b2):
    return pl.pallas_call(
        _mlp_body,
        out_shape=jax.ShapeDtypeStruct((GT, L), jnp.float32),
    )(sums, cnts, W1, b1, prelu_a, W2, b2)


BLKC = 112
NBC = RW // BLKC

@functools.partial(
    pl.kernel,
    out_type=jax.ShapeDtypeStruct((N * D,), jnp.float32),
    mesh=_mesh,
    scratch_types=[
        pltpu.VMEM((BLKC * D,), jnp.float32),
        pltpu.VMEM((BLKC * D,), jnp.float32),
        pltpu.VMEM((BLKC * D,), jnp.float32),
        pltpu.VMEM((BLKC * D,), jnp.float32),
        pltpu.VMEM((BLKC,), jnp.int32),
        pltpu.VMEM((BLKC,), jnp.int32),
        pltpu.VMEM((G * L,), jnp.float32),
        pltpu.SemaphoreType.DMA,
        pltpu.SemaphoreType.DMA,
        pltpu.SemaphoreType.DMA,
        pltpu.SemaphoreType.DMA,
    ],
)
def _scale_nodes(x_hbm, b_hbm, s_hbm, out_hbm,
                 in0, in1, out0, out1, ids0, ids1, sv,
                 semL0, semL1, semS0, semS1):
    sid = lax.axis_index("s")
    cid = lax.axis_index("c")
    wid = cid * NS + sid
    pltpu.sync_copy(s_hbm.at[pl.ds(0, G * L)], sv)
    base0 = wid * RW

    def _base(t):
        return jnp.minimum(base0 + t * BLKC, N - BLKC)

    def _start_load(t, inb, idb, sem):
        b = _base(t)
        pltpu.async_copy(x_hbm.at[pl.ds(b * D, BLKC * D)], inb, sem)
        pltpu.async_copy(b_hbm.at[pl.ds(b, BLKC)], idb, sem)

    def _wait_load(t, inb, idb, sem):
        b = _base(t)
        pltpu.make_async_copy(x_hbm.at[pl.ds(b * D, BLKC * D)], inb,
                              sem).wait()
        pltpu.make_async_copy(b_hbm.at[pl.ds(b, BLKC)], idb, sem).wait()

    def _start_store(t, outb, sem):
        b = _base(t)
        pltpu.async_copy(outb, out_hbm.at[pl.ds(b * D, BLKC * D)], sem)

    def _wait_store(t, outb, sem):
        b = _base(t)
        pltpu.make_async_copy(outb, out_hbm.at[pl.ds(b * D, BLKC * D)],
                              sem).wait()

    def _compute(inb, idb, outb):
        def _group_body(g, _):
            ids16 = idb[pl.ds(g * L, L)]
            for j in range(L):
                rid = ids16[j]
                srow = sv[pl.ds(rid * L, L)]
                o = (g * L + j) * D
                for k in range(DC):
                    outb[pl.ds(o + k * L, L)] = (
                        inb[pl.ds(o + k * L, L)] * srow)
            return 0
        lax.fori_loop(0, BLKC // L, _group_body, 0)

    _start_load(0, in0, ids0, semL0)

    def _pair_body(p, _):
        t0 = 2 * p
        _start_load(t0 + 1, in1, ids1, semL1)
        _wait_load(t0, in0, ids0, semL0)

        @pl.when(p >= 1)
        def _():
            _wait_store(t0 - 2, out0, semS0)

        _compute(in0, ids0, out0)
        _start_store(t0, out0, semS0)

        @pl.when(p < NBC // 2 - 1)
        def _():
            _start_load(t0 + 2, in0, ids0, semL0)

        _wait_load(t0 + 1, in1, ids1, semL1)

        @pl.when(p >= 1)
        def _():
            _wait_store(t0 - 1, out1, semS1)

        _compute(in1, ids1, out1)
        _start_store(t0 + 1, out1, semS1)
        return 0

    lax.fori_loop(0, NBC // 2, _pair_body, 0)
    _wait_store(NBC - 2, out0, semS0)
    _wait_store(NBC - 1, out1, semS1)


def kernel(x, batch, W1, b1, prelu_a, W2, b2):
    bi = batch.astype(jnp.int32)
    xf = jnp.reshape(x, (-1,))
    sums, cnts = _segment_sums(xf, bi)
    scores = _attention_scores(
        jnp.reshape(sums, (NC, GT, D)),
        jnp.reshape(cnts, (NC, GT, L)),
        W1,
        jnp.reshape(b1, (1, -1)),
        jnp.reshape(jnp.asarray(prelu_a, jnp.float32), (1, 1)),
        jnp.tile(W2, (L, 1)),
        jnp.reshape(b2, (1, 1)),
    )
    return jnp.reshape(_scale_nodes(xf, bi, jnp.reshape(scores, (-1,))),
                       (N, D))


# speedup vs baseline: 6.5404x; 1.1482x over previous
import functools

import jax
import jax.numpy as jnp
from jax import lax
from jax.experimental import pallas as pl
from jax.experimental.pallas import tpu as pltpu
from jax.experimental.pallas import tpu_sc as plsc

NC = 2
NS = 16
L = 16
NW = NC * NS

N = 100000
D = 128
DC = D // L
G = 1024
GT = 1152
GROWS = GT // NS

RW = 3136
BLK = 224
NBLK = RW // BLK
TAIL_FULL = (N - 31 * RW) // BLK
TAIL_REM = (N - 31 * RW) % BLK

_mesh = plsc.VectorSubcoreMesh(core_axis_name="c", subcore_axis_name="s")


@functools.partial(
    pl.kernel,
    out_type=(
        jax.ShapeDtypeStruct((NC, GT * D), jnp.float32),
        jax.ShapeDtypeStruct((NC, GT * L), jnp.float32),
    ),
    mesh=_mesh,
    scratch_types=[
        pltpu.VMEM((BLK * D,), jnp.float32),
        pltpu.VMEM((BLK * D,), jnp.float32),
        pltpu.VMEM((RW + L,), jnp.int32),
        pltpu.VMEM((D,), jnp.float32),
        pltpu.VMEM((L,), jnp.float32),
        pltpu.VMEM((D,), jnp.int32),
        pltpu.VMEM((L,), jnp.int32),
        pltpu.VMEM((GROWS * D,), jnp.float32),
        pltpu.VMEM((GROWS * L,), jnp.float32),
        pltpu.VMEM_SHARED((GT * D,), jnp.float32),
        pltpu.VMEM_SHARED((GT * L,), jnp.float32),
        pltpu.SemaphoreType.DMA,
        pltpu.SemaphoreType.DMA,
        [pltpu.VMEM((D,), jnp.int32)] * 4,
        [pltpu.SemaphoreType.DMA] * 4,
        pltpu.VMEM((D,), jnp.float32),
        pltpu.VMEM((L,), jnp.float32),
        pltpu.VMEM((D,), jnp.int32),
        pltpu.VMEM((L,), jnp.int32),
        pltpu.SemaphoreType.DMA,
    ],
)
def _segment_sums(x_hbm, b_hbm, sums_hbm, cnts_hbm,
                  xb0, xb1, ib, stage_row, stage_cnt, stage_ri, stage_ci,
                  obuf, cbuf, sums_sh, cnts_sh, sem0, sem1, zri, zsem,
                  fbuf, fcnt, fri, fci, fsem):
    sid = lax.axis_index("s")
    cid = lax.axis_index("c")
    wid = cid * NS + sid
    lane = lax.iota(jnp.int32, L)
    zv = jnp.zeros((L,), jnp.float32)

    for k in range(DC):
        stage_row[pl.ds(k * L, L)] = zv

    NZS = G * D // NS // D
    NZC = G * L // NS // D

    def _zloop(it, _):
        for slot in range(4):
            i = it * 4 + slot

            @pl.when(it >= 1)
            def _():
                pltpu.make_async_copy(stage_row, sums_sh.at[zri[slot]],
                                      zsem[slot]).wait()

            is_sum = i < NZS
            base = jnp.where(is_sum, sid * (G * D // NS) + i * D,
                             sid * (G * L // NS) + (i - NZS) * D)
            for k in range(DC):
                zri[slot][pl.ds(k * L, L)] = base + k * L + lane

            @pl.when(is_sum)
            def _():
                pltpu.async_copy(stage_row, sums_sh.at[zri[slot]],
                                 zsem[slot])

            @pl.when(jnp.logical_not(is_sum))
            def _():
                pltpu.async_copy(stage_row, cnts_sh.at[zri[slot]],
                                 zsem[slot])
        return 0
    lax.fori_loop(0, (NZS + NZC) // 4, _zloop, 0)
    for slot in range(4):
        pltpu.make_async_copy(stage_row, sums_sh.at[zri[slot]],
                              zsem[slot]).wait()

    for k in range(DC):
        fri[pl.ds(k * L, L)] = k * L + lane
        fbuf[pl.ds(k * L, L)] = zv
    fci[pl.ds(0, L)] = lane
    fcnt[pl.ds(0, L)] = zv
    pltpu.async_copy(fbuf, sums_sh.at[fri], fsem, add=True)
    pltpu.async_copy(fcnt, cnts_sh.at[fci], fsem, add=True)
    plsc.subcore_barrier()

    def _drain_flush():
        pltpu.make_async_copy(fbuf, sums_sh.at[fri], fsem).wait()
        pltpu.make_async_copy(fcnt, cnts_sh.at[fci], fsem).wait()

    def _flush(cnt, cur):
        _drain_flush()
        base_s = cur * D
        for k in range(DC):
            fri[pl.ds(k * L, L)] = base_s + (k * L) + lane
            fbuf[pl.ds(k * L, L)] = stage_row[pl.ds(k * L, L)]
        fcnt[pl.ds(0, L)] = jnp.full((L,), cnt, jnp.float32)
        fci[pl.ds(0, L)] = cur * L + lane
        pltpu.async_copy(fbuf, sums_sh.at[fri], fsem, add=True)
        pltpu.async_copy(fcnt, cnts_sh.at[fci], fsem, add=True)

    def _make_group_body(buf, t):
        def _group_body(g, carry):
            ids16 = ib[pl.ds(t * BLK + g * L, L)]
            uniform = ids16[L - 1] == carry[1]

            def _fast(carry):
                cnt, cur = carry
                for k in range(DC):
                    v = [buf[pl.ds((g * L + j) * D + k * L, L)]
                         for j in range(L)]
                    while len(v) > 1:
                        v = [v[i] + v[i + 1] for i in range(0, len(v), 2)]
                    o = k * L
                    stage_row[pl.ds(o, L)] = stage_row[pl.ds(o, L)] + v[0]
                return (cnt + jnp.float32(L), cur)

            def _slow(carry):
                accs0 = tuple(stage_row[pl.ds(k * L, L)] for k in range(DC))

                def _row_body(j, c):
                    cnt, cur = c[0], c[1]
                    accs = c[2:]
                    rid = ib[pl.ds(t * BLK + g * L + j, L)][0]
                    change = rid != cur

                    @pl.when(change)
                    def _(cnt=cnt, cur=cur, accs=accs):
                        _drain_flush()
                        for k in range(DC):
                            fri[pl.ds(k * L, L)] = cur * D + k * L + lane
                            fbuf[pl.ds(k * L, L)] = accs[k]
                        fcnt[pl.ds(0, L)] = jnp.full((L,), cnt, jnp.float32)
                        fci[pl.ds(0, L)] = cur * L + lane
                        pltpu.async_copy(fbuf, sums_sh.at[fri], fsem,
                                         add=True)
                        pltpu.async_copy(fcnt, cnts_sh.at[fci], fsem,
                                         add=True)

                    r = g * L + j
                    naccs = []
                    for k in range(DC):
                        row = buf[pl.ds(r * D + k * L, L)]
                        naccs.append(jnp.where(change, row, accs[k] + row))
                    ncnt = jnp.where(change, jnp.float32(1.0),
                                     cnt + jnp.float32(1.0))
                    return (ncnt, rid, *naccs)

                out = lax.fori_loop(0, L, _row_body,
                                    (carry[0], carry[1], *accs0))
                for k in range(DC):
                    stage_row[pl.ds(k * L, L)] = out[2 + k]
                return (out[0], out[1])

            return lax.cond(uniform, _fast, _slow, carry)
        return _group_body

    def _run_span(base0, nblk, tail_rows):
        nids = nblk * BLK + tail_rows
        pltpu.sync_copy(b_hbm.at[pl.ds(base0, nids)], ib.at[pl.ds(0, nids)])

        def _x_slice(t):
            return x_hbm.at[pl.ds((base0 + t * BLK) * D, BLK * D)]

        pltpu.async_copy(_x_slice(0), xb0, sem0)
        carry0 = (jnp.float32(0.0), jnp.int32(0))

        def _pair_body(p, carry):
            t0 = 2 * p
            pltpu.async_copy(_x_slice(t0 + 1), xb1, sem1)
            pltpu.make_async_copy(_x_slice(t0), xb0, sem0).wait()
            carry = lax.fori_loop(0, BLK // L,
                                  _make_group_body(xb0, t0), carry)

            @pl.when(p < nblk // 2 - 1)
            def _():
                pltpu.async_copy(_x_slice(t0 + 2), xb0, sem0)

            pltpu.make_async_copy(_x_slice(t0 + 1), xb1, sem1).wait()
            carry = lax.fori_loop(0, BLK // L,
                                  _make_group_body(xb1, t0 + 1), carry)
            return carry

        carry = lax.fori_loop(0, nblk // 2, _pair_body, carry0)
        if tail_rows:
            base = base0 + nblk * BLK
            pltpu.sync_copy(x_hbm.at[pl.ds(base * D, tail_rows * D)],
                            xb0.at[pl.ds(0, tail_rows * D)])
            carry = lax.fori_loop(0, tail_rows // L,
                                  _make_group_body(xb0, nblk), carry)
        cnt, cur = carry
        _flush(cnt, cur)
        _drain_flush()

    @pl.when(wid < NW - 1)
    def _():
        _run_span(wid * RW, NBLK, 0)

    @pl.when(wid == NW - 1)
    def _():
        _run_span((NW - 1) * RW, TAIL_FULL, TAIL_REM)

    plsc.subcore_barrier()

    for j in range(NS):
        @pl.when(sid == j)
        def _(j=j):
            pltpu.sync_copy(sums_sh.at[pl.ds(j * GROWS * D, GROWS * D)], obuf)
            pltpu.sync_copy(obuf,
                            sums_hbm.at[cid, pl.ds(j * GROWS * D, GROWS * D)])
            pltpu.sync_copy(cnts_sh.at[pl.ds(j * GROWS * L, GROWS * L)], cbuf)
            pltpu.sync_copy(cbuf,
                            cnts_hbm.at[cid, pl.ds(j * GROWS * L, GROWS * L)])


def _mlp_body(sums_ref, cnts_ref, w1_ref, b1_ref, a_ref, w2_ref, b2_ref,
              out_ref):
    total = sums_ref[0] + sums_ref[1]
    cnt = cnts_ref[0] + cnts_ref[1]
    cnt1 = jnp.maximum(cnt[:, 0:1], 1.0)
    mean = total / cnt1
    h = lax.dot_general(mean, w1_ref[...], (((1,), (1,)), ((), ())),
                        preferred_element_type=jnp.float32)
    h = h + b1_ref[...]
    h = jnp.where(h >= 0, h, a_ref[...] * h)
    s = lax.dot_general(h, w2_ref[...], (((1,), (1,)), ((), ())),
                        preferred_element_type=jnp.float32)
    out_ref[...] = jax.nn.sigmoid(s + b2_ref[...])


def _attention_scores(sums, cnts, W1, b1, prelu_a, W2, b2):
    return pl.pallas_call(
        _mlp_body,
        out_shape=jax.ShapeDtypeStruct((GT, L), jnp.float32),
    )(sums, cnts, W1, b1, prelu_a, W2, b2)


BLKC = 112
NBC = RW // BLKC

@functools.partial(
    pl.kernel,
    out_type=jax.ShapeDtypeStruct((N * D,), jnp.float32),
    mesh=_mesh,
    scratch_types=[
        pltpu.VMEM((BLKC * D,), jnp.float32),
        pltpu.VMEM((BLKC * D,), jnp.float32),
        pltpu.VMEM((BLKC * D,), jnp.float32),
        pltpu.VMEM((BLKC * D,), jnp.float32),
        pltpu.VMEM((BLKC,), jnp.int32),
        pltpu.VMEM((BLKC,), jnp.int32),
        pltpu.VMEM((G * L,), jnp.float32),
        pltpu.SemaphoreType.DMA,
        pltpu.SemaphoreType.DMA,
        pltpu.SemaphoreType.DMA,
        pltpu.SemaphoreType.DMA,
    ],
)
def _scale_nodes(x_hbm, b_hbm, s_hbm, out_hbm,
                 in0, in1, out0, out1, ids0, ids1, sv,
                 semL0, semL1, semS0, semS1):
    sid = lax.axis_index("s")
    cid = lax.axis_index("c")
    wid = cid * NS + sid
    pltpu.sync_copy(s_hbm.at[pl.ds(0, G * L)], sv)
    base0 = wid * RW

    def _base(t):
        return jnp.minimum(base0 + t * BLKC, N - BLKC)

    def _start_load(t, inb, idb, sem):
        b = _base(t)
        pltpu.async_copy(x_hbm.at[pl.ds(b * D, BLKC * D)], inb, sem)
        pltpu.async_copy(b_hbm.at[pl.ds(b, BLKC)], idb, sem)

    def _wait_load(t, inb, idb, sem):
        b = _base(t)
        pltpu.make_async_copy(x_hbm.at[pl.ds(b * D, BLKC * D)], inb,
                              sem).wait()
        pltpu.make_async_copy(b_hbm.at[pl.ds(b, BLKC)], idb, sem).wait()

    def _start_store(t, outb, sem):
        b = _base(t)
        pltpu.async_copy(outb, out_hbm.at[pl.ds(b * D, BLKC * D)], sem)

    def _wait_store(t, outb, sem):
        b = _base(t)
        pltpu.make_async_copy(outb, out_hbm.at[pl.ds(b * D, BLKC * D)],
                              sem).wait()

    def _compute(inb, idb, outb):
        def _group_body(g, _):
            ids16 = idb[pl.ds(g * L, L)]
            for j in range(L):
                rid = ids16[j]
                srow = sv[pl.ds(rid * L, L)]
                o = (g * L + j) * D
                for k in range(DC):
                    outb[pl.ds(o + k * L, L)] = (
                        inb[pl.ds(o + k * L, L)] * srow)
            return 0
        lax.fori_loop(0, BLKC // L, _group_body, 0)

    _start_load(0, in0, ids0, semL0)

    def _pair_body(p, _):
        t0 = 2 * p
        _start_load(t0 + 1, in1, ids1, semL1)
        _wait_load(t0, in0, ids0, semL0)

        @pl.when(p >= 1)
        def _():
            _wait_store(t0 - 2, out0, semS0)

        _compute(in0, ids0, out0)
        _start_store(t0, out0, semS0)

        @pl.when(p < NBC // 2 - 1)
        def _():
            _start_load(t0 + 2, in0, ids0, semL0)

        _wait_load(t0 + 1, in1, ids1, semL1)

        @pl.when(p >= 1)
        def _():
            _wait_store(t0 - 1, out1, semS1)

        _compute(in1, ids1, out1)
        _start_store(t0 + 1, out1, semS1)
        return 0

    lax.fori_loop(0, NBC // 2, _pair_body, 0)
    _wait_store(NBC - 2, out0, semS0)
    _wait_store(NBC - 1, out1, semS1)


def kernel(x, batch, W1, b1, prelu_a, W2, b2):
    bi = batch.astype(jnp.int32)
    xf = jnp.reshape(x, (-1,))
    sums, cnts = _segment_sums(xf, bi)
    scores = _attention_scores(
        jnp.reshape(sums, (NC, GT, D)),
        jnp.reshape(cnts, (NC, GT, L)),
        W1,
        jnp.reshape(b1, (1, -1)),
        jnp.reshape(jnp.asarray(prelu_a, jnp.float32), (1, 1)),
        jnp.tile(W2, (L, 1)),
        jnp.reshape(b2, (1, 1)),
    )
    return jnp.reshape(_scale_nodes(xf, bi, jnp.reshape(scores, (-1,))),
                       (N, D))


# speedup vs baseline: 6.5559x; 1.0024x over previous
import functools

import jax
import jax.numpy as jnp
from jax import lax
from jax.experimental import pallas as pl
from jax.experimental.pallas import tpu as pltpu
from jax.experimental.pallas import tpu_sc as plsc

NC = 2
NS = 16
L = 16
NW = NC * NS

N = 100000
D = 128
DC = D // L
G = 1024
GT = 1152
GROWS = GT // NS

RW = 3136
BLK = 224
NBLK = RW // BLK
TAIL_FULL = (N - 31 * RW) // BLK
TAIL_REM = (N - 31 * RW) % BLK

_mesh = plsc.VectorSubcoreMesh(core_axis_name="c", subcore_axis_name="s")


@functools.partial(
    pl.kernel,
    out_type=(
        jax.ShapeDtypeStruct((NC, GT * D), jnp.float32),
        jax.ShapeDtypeStruct((NC, GT * L), jnp.float32),
    ),
    mesh=_mesh,
    scratch_types=[
        pltpu.VMEM((BLK * D,), jnp.float32),
        pltpu.VMEM((BLK * D,), jnp.float32),
        pltpu.VMEM((RW + L,), jnp.int32),
        pltpu.VMEM((D,), jnp.float32),
        pltpu.VMEM((L,), jnp.float32),
        pltpu.VMEM((D,), jnp.int32),
        pltpu.VMEM((L,), jnp.int32),
        pltpu.VMEM((GROWS * D,), jnp.float32),
        pltpu.VMEM((GROWS * L,), jnp.float32),
        pltpu.VMEM_SHARED((GT * D,), jnp.float32),
        pltpu.VMEM_SHARED((GT * L,), jnp.float32),
        pltpu.SemaphoreType.DMA,
        pltpu.SemaphoreType.DMA,
        [pltpu.VMEM((D,), jnp.int32)] * 4,
        [pltpu.SemaphoreType.DMA] * 4,
        pltpu.VMEM((D,), jnp.float32),
        pltpu.VMEM((L,), jnp.float32),
        pltpu.VMEM((D,), jnp.int32),
        pltpu.VMEM((L,), jnp.int32),
        pltpu.SemaphoreType.DMA,
    ],
)
def _segment_sums(x_hbm, b_hbm, sums_hbm, cnts_hbm,
                  xb0, xb1, ib, stage_row, stage_cnt, stage_ri, stage_ci,
                  obuf, cbuf, sums_sh, cnts_sh, sem0, sem1, zri, zsem,
                  fbuf, fcnt, fri, fci, fsem):
    sid = lax.axis_index("s")
    cid = lax.axis_index("c")
    wid = cid * NS + sid
    lane = lax.iota(jnp.int32, L)
    zv = jnp.zeros((L,), jnp.float32)

    for k in range(DC):
        stage_row[pl.ds(k * L, L)] = zv

    NZS = G * D // NS // D
    NZC = G * L // NS // D

    def _zloop(it, _):
        for slot in range(4):
            i = it * 4 + slot

            @pl.when(it >= 1)
            def _():
                pltpu.make_async_copy(stage_row, sums_sh.at[zri[slot]],
                                      zsem[slot]).wait()

            is_sum = i < NZS
            base = jnp.where(is_sum, sid * (G * D // NS) + i * D,
                             sid * (G * L // NS) + (i - NZS) * D)
            for k in range(DC):
                zri[slot][pl.ds(k * L, L)] = base + k * L + lane

            @pl.when(is_sum)
            def _():
                pltpu.async_copy(stage_row, sums_sh.at[zri[slot]],
                                 zsem[slot])

            @pl.when(jnp.logical_not(is_sum))
            def _():
                pltpu.async_copy(stage_row, cnts_sh.at[zri[slot]],
                                 zsem[slot])
        return 0
    lax.fori_loop(0, (NZS + NZC) // 4, _zloop, 0)
    for slot in range(4):
        pltpu.make_async_copy(stage_row, sums_sh.at[zri[slot]],
                              zsem[slot]).wait()

    for k in range(DC):
        fri[pl.ds(k * L, L)] = k * L + lane
        fbuf[pl.ds(k * L, L)] = zv
    fci[pl.ds(0, L)] = lane
    fcnt[pl.ds(0, L)] = zv
    pltpu.async_copy(fbuf, sums_sh.at[fri], fsem, add=True)
    pltpu.async_copy(fcnt, cnts_sh.at[fci], fsem, add=True)
    plsc.subcore_barrier()

    def _drain_flush():
        pltpu.make_async_copy(fbuf, sums_sh.at[fri], fsem).wait()
        pltpu.make_async_copy(fcnt, cnts_sh.at[fci], fsem).wait()

    def _flush(cnt, cur):
        _drain_flush()
        base_s = cur * D
        for k in range(DC):
            fri[pl.ds(k * L, L)] = base_s + (k * L) + lane
            fbuf[pl.ds(k * L, L)] = stage_row[pl.ds(k * L, L)]
        fcnt[pl.ds(0, L)] = jnp.full((L,), cnt, jnp.float32)
        fci[pl.ds(0, L)] = cur * L + lane
        pltpu.async_copy(fbuf, sums_sh.at[fri], fsem, add=True)
        pltpu.async_copy(fcnt, cnts_sh.at[fci], fsem, add=True)

    def _make_group_body(buf, t):
        def _group_body(g, carry):
            ids16 = ib[pl.ds(t * BLK + g * L, L)]
            uniform = ids16[L - 1] == carry[1]

            def _fast(carry):
                cnt, cur = carry
                for k in range(DC):
                    v = [buf[pl.ds((g * L + j) * D + k * L, L)]
                         for j in range(L)]
                    while len(v) > 1:
                        v = [v[i] + v[i + 1] for i in range(0, len(v), 2)]
                    o = k * L
                    stage_row[pl.ds(o, L)] = stage_row[pl.ds(o, L)] + v[0]
                return (cnt + jnp.float32(L), cur)

            def _slow(carry):
                accs0 = tuple(stage_row[pl.ds(k * L, L)] for k in range(DC))

                def _one_row(j, cnt, cur, accs):
                    rid = ib[pl.ds(t * BLK + g * L + j, L)][0]
                    change = rid != cur

                    @pl.when(change)
                    def _(cnt=cnt, cur=cur, accs=accs):
                        _drain_flush()
                        for k in range(DC):
                            fri[pl.ds(k * L, L)] = cur * D + k * L + lane
                            fbuf[pl.ds(k * L, L)] = accs[k]
                        fcnt[pl.ds(0, L)] = jnp.full((L,), cnt, jnp.float32)
                        fci[pl.ds(0, L)] = cur * L + lane
                        pltpu.async_copy(fbuf, sums_sh.at[fri], fsem,
                                         add=True)
                        pltpu.async_copy(fcnt, cnts_sh.at[fci], fsem,
                                         add=True)

                    r = g * L + j
                    naccs = []
                    for k in range(DC):
                        row = buf[pl.ds(r * D + k * L, L)]
                        naccs.append(jnp.where(change, row, accs[k] + row))
                    ncnt = jnp.where(change, jnp.float32(1.0),
                                     cnt + jnp.float32(1.0))
                    return ncnt, rid, naccs

                def _row_body(jj, c):
                    cnt, cur = c[0], c[1]
                    accs = list(c[2:])
                    for u in range(2):
                        cnt, cur, accs = _one_row(2 * jj + u, cnt, cur, accs)
                    return (cnt, cur, *accs)

                out = lax.fori_loop(0, L // 2, _row_body,
                                    (carry[0], carry[1], *accs0))
                for k in range(DC):
                    stage_row[pl.ds(k * L, L)] = out[2 + k]
                return (out[0], out[1])

            return lax.cond(uniform, _fast, _slow, carry)
        return _group_body

    def _run_span(base0, nblk, tail_rows):
        nids = nblk * BLK + tail_rows
        pltpu.sync_copy(b_hbm.at[pl.ds(base0, nids)], ib.at[pl.ds(0, nids)])

        def _x_slice(t):
            return x_hbm.at[pl.ds((base0 + t * BLK) * D, BLK * D)]

        pltpu.async_copy(_x_slice(0), xb0, sem0)
        carry0 = (jnp.float32(0.0), jnp.int32(0))

        def _pair_body(p, carry):
            t0 = 2 * p
            pltpu.async_copy(_x_slice(t0 + 1), xb1, sem1)
            pltpu.make_async_copy(_x_slice(t0), xb0, sem0).wait()
            carry = lax.fori_loop(0, BLK // L,
                                  _make_group_body(xb0, t0), carry)

            @pl.when(p < nblk // 2 - 1)
            def _():
                pltpu.async_copy(_x_slice(t0 + 2), xb0, sem0)

            pltpu.make_async_copy(_x_slice(t0 + 1), xb1, sem1).wait()
            carry = lax.fori_loop(0, BLK // L,
                                  _make_group_body(xb1, t0 + 1), carry)
            return carry

        carry = lax.fori_loop(0, nblk // 2, _pair_body, carry0)
        if tail_rows:
            base = base0 + nblk * BLK
            pltpu.sync_copy(x_hbm.at[pl.ds(base * D, tail_rows * D)],
                            xb0.at[pl.ds(0, tail_rows * D)])
            carry = lax.fori_loop(0, tail_rows // L,
                                  _make_group_body(xb0, nblk), carry)
        cnt, cur = carry
        _flush(cnt, cur)
        _drain_flush()

    @pl.when(wid < NW - 1)
    def _():
        _run_span(wid * RW, NBLK, 0)

    @pl.when(wid == NW - 1)
    def _():
        _run_span((NW - 1) * RW, TAIL_FULL, TAIL_REM)

    plsc.subcore_barrier()

    for j in range(NS):
        @pl.when(sid == j)
        def _(j=j):
            pltpu.sync_copy(sums_sh.at[pl.ds(j * GROWS * D, GROWS * D)], obuf)
            pltpu.sync_copy(obuf,
                            sums_hbm.at[cid, pl.ds(j * GROWS * D, GROWS * D)])
            pltpu.sync_copy(cnts_sh.at[pl.ds(j * GROWS * L, GROWS * L)], cbuf)
            pltpu.sync_copy(cbuf,
                            cnts_hbm.at[cid, pl.ds(j * GROWS * L, GROWS * L)])


def _mlp_body(sums_ref, cnts_ref, w1_ref, b1_ref, a_ref, w2_ref, b2_ref,
              out_ref):
    total = sums_ref[0] + sums_ref[1]
    cnt = cnts_ref[0] + cnts_ref[1]
    cnt1 = jnp.maximum(cnt[:, 0:1], 1.0)
    mean = total / cnt1
    h = lax.dot_general(mean, w1_ref[...], (((1,), (1,)), ((), ())),
                        preferred_element_type=jnp.float32)
    h = h + b1_ref[...]
    h = jnp.where(h >= 0, h, a_ref[...] * h)
    s = lax.dot_general(h, w2_ref[...], (((1,), (1,)), ((), ())),
                        preferred_element_type=jnp.float32)
    out_ref[...] = jax.nn.sigmoid(s + b2_ref[...])


def _attention_scores(sums, cnts, W1, b1, prelu_a, W2, b2):
    return pl.pallas_call(
        _mlp_body,
        out_shape=jax.ShapeDtypeStruct((GT, L), jnp.float32),
    )(sums, cnts, W1, b1, prelu_a, W2, b2)


BLKC = 112
NBC = RW // BLKC

@functools.partial(
    pl.kernel,
    out_type=jax.ShapeDtypeStruct((N * D,), jnp.float32),
    mesh=_mesh,
    scratch_types=[
        pltpu.VMEM((BLKC * D,), jnp.float32),
        pltpu.VMEM((BLKC * D,), jnp.float32),
        pltpu.VMEM((BLKC * D,), jnp.float32),
        pltpu.VMEM((BLKC * D,), jnp.float32),
        pltpu.VMEM((BLKC,), jnp.int32),
        pltpu.VMEM((BLKC,), jnp.int32),
        pltpu.VMEM((G * L,), jnp.float32),
        pltpu.SemaphoreType.DMA,
        pltpu.SemaphoreType.DMA,
        pltpu.SemaphoreType.DMA,
        pltpu.SemaphoreType.DMA,
    ],
)
def _scale_nodes(x_hbm, b_hbm, s_hbm, out_hbm,
                 in0, in1, out0, out1, ids0, ids1, sv,
                 semL0, semL1, semS0, semS1):
    sid = lax.axis_index("s")
    cid = lax.axis_index("c")
    wid = cid * NS + sid
    pltpu.sync_copy(s_hbm.at[pl.ds(0, G * L)], sv)
    base0 = wid * RW

    def _base(t):
        return jnp.minimum(base0 + t * BLKC, N - BLKC)

    def _start_load(t, inb, idb, sem):
        b = _base(t)
        pltpu.async_copy(x_hbm.at[pl.ds(b * D, BLKC * D)], inb, sem)
        pltpu.async_copy(b_hbm.at[pl.ds(b, BLKC)], idb, sem)

    def _wait_load(t, inb, idb, sem):
        b = _base(t)
        pltpu.make_async_copy(x_hbm.at[pl.ds(b * D, BLKC * D)], inb,
                              sem).wait()
        pltpu.make_async_copy(b_hbm.at[pl.ds(b, BLKC)], idb, sem).wait()

    def _start_store(t, outb, sem):
        b = _base(t)
        pltpu.async_copy(outb, out_hbm.at[pl.ds(b * D, BLKC * D)], sem)

    def _wait_store(t, outb, sem):
        b = _base(t)
        pltpu.make_async_copy(outb, out_hbm.at[pl.ds(b * D, BLKC * D)],
                              sem).wait()

    def _compute(inb, idb, outb):
        def _group_body(g, _):
            ids16 = idb[pl.ds(g * L, L)]
            for j in range(L):
                rid = ids16[j]
                srow = sv[pl.ds(rid * L, L)]
                o = (g * L + j) * D
                for k in range(DC):
                    outb[pl.ds(o + k * L, L)] = (
                        inb[pl.ds(o + k * L, L)] * srow)
            return 0
        lax.fori_loop(0, BLKC // L, _group_body, 0)

    _start_load(0, in0, ids0, semL0)

    def _pair_body(p, _):
        t0 = 2 * p
        _start_load(t0 + 1, in1, ids1, semL1)
        _wait_load(t0, in0, ids0, semL0)

        @pl.when(p >= 1)
        def _():
            _wait_store(t0 - 2, out0, semS0)

        _compute(in0, ids0, out0)
        _start_store(t0, out0, semS0)

        @pl.when(p < NBC // 2 - 1)
        def _():
            _start_load(t0 + 2, in0, ids0, semL0)

        _wait_load(t0 + 1, in1, ids1, semL1)

        @pl.when(p >= 1)
        def _():
            _wait_store(t0 - 1, out1, semS1)

        _compute(in1, ids1, out1)
        _start_store(t0 + 1, out1, semS1)
        return 0

    lax.fori_loop(0, NBC // 2, _pair_body, 0)
    _wait_store(NBC - 2, out0, semS0)
    _wait_store(NBC - 1, out1, semS1)


def kernel(x, batch, W1, b1, prelu_a, W2, b2):
    bi = batch.astype(jnp.int32)
    xf = jnp.reshape(x, (-1,))
    sums, cnts = _segment_sums(xf, bi)
    scores = _attention_scores(
        jnp.reshape(sums, (NC, GT, D)),
        jnp.reshape(cnts, (NC, GT, L)),
        W1,
        jnp.reshape(b1, (1, -1)),
        jnp.reshape(jnp.asarray(prelu_a, jnp.float32), (1, 1)),
        jnp.tile(W2, (L, 1)),
        jnp.reshape(b2, (1, 1)),
    )
    return jnp.reshape(_scale_nodes(xf, bi, jnp.reshape(scores, (-1,))),
                       (N, D))
